# baseline (device time: 487090 ns/iter reference)
import jax
import jax.numpy as jnp
import numpy as np
from jax import lax
from jax.experimental import pallas as pl
from jax.experimental.pallas import tpu as pltpu

N_DEV = 4
N_HOPS = 1
T_CORR = 256
NDB = 8
TT = 8


def kernel(x, A, B, C):
    Bb, S, D = x.shape
    N = A.shape[1]
    DB = D // NDB
    At = A.T

    def body(x_ref, at_ref, b_ref, c_ref, y_ref,
             bt_ref, ct_ref, comm_ref, send_sem, recv_sem):
        my = lax.axis_index("i")
        left = lax.rem(my + (N_DEV - 1), N_DEV)
        right = lax.rem(my + 1, N_DEV)

        dA = jnp.exp(at_ref[...])

        def repack(tc, _):
            b_ch = b_ref[:, pl.ds(tc * TT, TT), :]
            c_ch = c_ref[:, pl.ds(tc * TT, TT), :]
            bt_ref[:, pl.ds(tc, 1), :, :] = jnp.swapaxes(
                b_ch, 1, 2).reshape(Bb, 1, N, TT)
            ct_ref[:, pl.ds(tc, 1), :, :] = jnp.swapaxes(
                c_ch, 1, 2).reshape(Bb, 1, N, TT)
            return 0

        lax.fori_loop(0, S // TT, repack, 0)

        h_last_parts = []
        for db in range(NDB):
            dsl = slice(db * DB, (db + 1) * DB)
            dA_d = dA[:, dsl]

            def chunk(tc, h_d, dsl=dsl, dA_d=dA_d):
                x_ch = x_ref[:, pl.ds(tc * TT, TT), dsl]
                b_ch = bt_ref[:, pl.ds(tc, 1), :, :].reshape(Bb, N, TT)
                c_ch = ct_ref[:, pl.ds(tc, 1), :, :].reshape(Bb, N, TT)
                ys = []
                for tt in range(TT):
                    b_col = b_ch[:, :, tt:tt + 1]
                    c_col = c_ch[:, :, tt:tt + 1]
                    x_row = x_ch[:, tt:tt + 1, :]
                    h_d = h_d * dA_d[None] + b_col * x_row
                    ys.append(jnp.sum(h_d * c_col, axis=1, keepdims=True))
                y_ref[:, pl.ds(tc * TT, TT), dsl] = jnp.concatenate(ys, axis=1)
                return h_d

            h_last_parts.append(lax.fori_loop(
                0, S // TT, chunk, jnp.zeros((Bb, N, DB), jnp.float32)))

        comm_ref[0] = jnp.concatenate(h_last_parts, axis=2)

        barrier = pltpu.get_barrier_semaphore()
        for nbr in (left, right):
            pl.semaphore_signal(barrier, inc=1, device_id=(nbr,),
                                device_id_type=pl.DeviceIdType.MESH)
        pl.semaphore_wait(barrier, 2)

        rdma = pltpu.make_async_remote_copy(
            src_ref=comm_ref.at[0],
            dst_ref=comm_ref.at[1],
            send_sem=send_sem,
            recv_sem=recv_sem,
            device_id=(right,),
            device_id_type=pl.DeviceIdType.MESH)
        rdma.start()
        rdma.wait()

        m = jnp.where(my >= 1, np.float32(1), np.float32(0))
        for db in range(NDB):
            dsl = slice(db * DB, (db + 1) * DB)
            dA_d = dA[:, dsl]
            h0_d = m * comm_ref[1][:, :, dsl]

            def corr(tc, hc_d, dsl=dsl, dA_d=dA_d):
                c_ch = ct_ref[:, pl.ds(tc, 1), :, :].reshape(Bb, N, TT)
                ys = []
                for tt in range(TT):
                    hc_d = hc_d * dA_d[None]
                    c_col = c_ch[:, :, tt:tt + 1]
                    ys.append(jnp.sum(hc_d * c_col, axis=1, keepdims=True))
                idx = (slice(None), pl.ds(tc * TT, TT), dsl)
                y_ref[idx] = y_ref[idx] + jnp.concatenate(ys, axis=1)
                return hc_d

            lax.fori_loop(0, T_CORR // TT, corr, h0_d)

    return pl.pallas_call(
        body,
        out_shape=jax.ShapeDtypeStruct((Bb, S, D), jnp.float32),
        in_specs=[pl.BlockSpec(memory_space=pltpu.VMEM)] * 4,
        out_specs=pl.BlockSpec(memory_space=pltpu.VMEM),
        scratch_shapes=[
            pltpu.VMEM((Bb, S // TT, N, TT), jnp.float32),
            pltpu.VMEM((Bb, S // TT, N, TT), jnp.float32),
            pltpu.VMEM((2, Bb, N, D), jnp.float32),
            pltpu.SemaphoreType.DMA,
            pltpu.SemaphoreType.DMA,
        ],
        compiler_params=pltpu.CompilerParams(
            collective_id=0, vmem_limit_bytes=100 * 1024 * 1024),
    )(x, At, B, C)


# device time: 181035 ns/iter; 2.6906x vs baseline; 2.6906x over previous
import jax
import jax.numpy as jnp
import numpy as np
from jax import lax
from jax.experimental import pallas as pl
from jax.experimental.pallas import tpu as pltpu

N_DEV = 4

N_HOPS = 1
T_CORR = 256


def kernel(x, A, B, C):
    Bb, S, D = x.shape
    N = A.shape[1]
    At = A.T

    def body(x_ref, at_ref, b_ref, c_ref, y_ref, comm_ref, send_sems, recv_sems):
        my = lax.axis_index("i")
        left = lax.rem(my + (N_DEV - 1), N_DEV)
        right = lax.rem(my + 1, N_DEV)

        dA = jnp.exp(at_ref[...])

        def step(t, h):
            x_t = x_ref[:, pl.ds(t, 1), :]
            b_t = b_ref[:, pl.ds(t, 1), :]
            c_t = c_ref[:, pl.ds(t, 1), :]
            bT = jnp.swapaxes(b_t, 1, 2)
            h = h * dA[None] + bT * x_t
            y_t = lax.dot_general(
                c_t, h, (((2,), (1,)), ((0,), (0,))),
                preferred_element_type=jnp.float32)
            y_ref[:, pl.ds(t, 1), :] = y_t
            return h

        h_last = lax.fori_loop(0, S, step, jnp.zeros((Bb, N, D), jnp.float32),
                               unroll=8)

        comm_ref[0] = h_last

        barrier = pltpu.get_barrier_semaphore()
        for nbr in (left, right):
            pl.semaphore_signal(barrier, inc=1, device_id=(nbr,),
                                device_id_type=pl.DeviceIdType.MESH)
        pl.semaphore_wait(barrier, 2)

        for hop in range(N_HOPS):
            rdma = pltpu.make_async_remote_copy(
                src_ref=comm_ref.at[hop],
                dst_ref=comm_ref.at[hop + 1],
                send_sem=send_sems.at[hop],
                recv_sem=recv_sems.at[hop + 1],
                device_id=(right,),
                device_id_type=pl.DeviceIdType.MESH)
            rdma.start()
            rdma.wait()

        h0 = jnp.zeros((Bb, N, D), jnp.float32)
        w = jnp.ones((N, D), jnp.float32)
        dAL = jnp.exp(at_ref[...] * np.float32(S))
        for jj in range(1, N_HOPS + 1):
            m = jnp.where(my >= jj, np.float32(1), np.float32(0))
            h0 = h0 + (m * w)[None] * comm_ref[jj]
            w = w * dAL

        def cstep(t, hc):
            hc = hc * dA[None]
            c_t = c_ref[:, pl.ds(t, 1), :]
            y_c = lax.dot_general(
                c_t, hc, (((2,), (1,)), ((0,), (0,))),
                preferred_element_type=jnp.float32)
            y_ref[:, pl.ds(t, 1), :] = y_ref[:, pl.ds(t, 1), :] + y_c
            return hc

        lax.fori_loop(0, T_CORR, cstep, h0, unroll=8)

    return pl.pallas_call(
        body,
        out_shape=jax.ShapeDtypeStruct((Bb, S, D), jnp.float32),
        in_specs=[pl.BlockSpec(memory_space=pltpu.VMEM)] * 4,
        out_specs=pl.BlockSpec(memory_space=pltpu.VMEM),
        scratch_shapes=[
            pltpu.VMEM((N_DEV, Bb, N, D), jnp.float32),
            pltpu.SemaphoreType.DMA((N_DEV,)),
            pltpu.SemaphoreType.DMA((N_DEV,)),
        ],
        compiler_params=pltpu.CompilerParams(collective_id=0),
    )(x, At, B, C)
